# Initial kernel scaffold; baseline (speedup 1.0000x reference)
import jax
import jax.numpy as jnp
from jax.experimental import pallas as pl


def kernel(x, table):
    raise NotImplementedError("write your pallas kernel here")


# speedup vs baseline: 1.1607x; 1.1607x over previous
import jax
import jax.numpy as jnp
from jax import lax
from jax.experimental import pallas as pl
from jax.experimental.pallas import tpu as pltpu, tpu_sc as plsc

D = 32
NC = 2
NS = 16
L = 16
NW = NC * NS
CHUNK = 128
GROUPS = CHUNK // L


def _rsqrt(x):
    i = plsc.bitcast(x, jnp.int32)
    i = jnp.int32(0x5F3759DF) - (i >> 1)
    y = plsc.bitcast(i, jnp.float32)
    for _ in range(3):
        y = y * (1.5 - 0.5 * x * y * y)
    return y


def _body(x_hbm, table_hbm, out_hbm, idx_v, rows_v, out_v, sem):
    n_chunks = x_hbm.shape[1]
    per_w = n_chunks * CHUNK
    wid = lax.axis_index("s") * NC + lax.axis_index("c")
    pltpu.sync_copy(x_hbm.at[wid], idx_v)
    base = wid * per_w

    def chunk_body(c, carry):
        pltpu.async_copy(table_hbm.at[idx_v.at[c]], rows_v, sem).wait()

        def group_body(g, carry2):
            row_idx = g * L + lax.iota(jnp.int32, L)
            cols = []
            s = jnp.zeros((L,), jnp.float32)
            sq = jnp.zeros((L,), jnp.float32)
            for j in range(D):
                v = plsc.load_gather(
                    rows_v, [row_idx, jnp.full((L,), j, jnp.int32)])
                cols.append(v)
                s = s + v
                sq = sq + v * v
            mean = s * (1.0 / D)
            var = sq * (1.0 / D) - mean * mean
            rstd = _rsqrt(var + 1e-5)
            for j in range(D):
                y = (cols[j] - mean) * rstd
                plsc.store_scatter(
                    out_v, [row_idx, jnp.full((L,), j, jnp.int32)], y)
            return carry2

        lax.fori_loop(0, GROUPS, group_body, 0)
        pltpu.sync_copy(out_v, out_hbm.at[pl.ds(base + c * CHUNK, CHUNK)])
        return carry

    lax.fori_loop(0, n_chunks, chunk_body, 0)


def kernel(x, table):
    batch, fields = x.shape
    total = batch * fields
    n_chunks = total // (NW * CHUNK)
    x_re = x.reshape(NW, n_chunks, CHUNK).astype(jnp.int32)
    mesh = plsc.VectorSubcoreMesh(core_axis_name="c", subcore_axis_name="s")
    f = pl.kernel(
        _body,
        mesh=mesh,
        out_type=jax.ShapeDtypeStruct((total, D), jnp.float32),
        scratch_types=[
            pltpu.VMEM((n_chunks, CHUNK), jnp.int32),
            pltpu.VMEM((CHUNK, D), jnp.float32),
            pltpu.VMEM((CHUNK, D), jnp.float32),
            pltpu.SemaphoreType.DMA,
        ],
        compiler_params=pltpu.CompilerParams(
            needs_layout_passes=False, use_tc_tiling_on_sc=False),
    )
    out = f(x_re, table)
    return out.reshape(batch, fields, D)


# speedup vs baseline: 1.2671x; 1.0916x over previous
import jax
import jax.numpy as jnp
from jax import lax
from jax.experimental import pallas as pl
from jax.experimental.pallas import tpu as pltpu, tpu_sc as plsc

D = 32
NC = 2
NS = 16
L = 16
NW = NC * NS
GCHUNK = 128
KG = 4
SCHUNK = GCHUNK * KG
GROUPS = SCHUNK // L


def _rsqrt(x):
    i = plsc.bitcast(x, jnp.int32)
    i = jnp.int32(0x5F3759DF) - (i >> 1)
    y = plsc.bitcast(i, jnp.float32)
    for _ in range(3):
        y = y * (1.5 - 0.5 * x * y * y)
    return y


def _body(x_hbm, table_hbm, out_hbm, idx_v, rows0, rows1, out0, out1,
          gsem0, gsem1, osem0, osem1):
    n_sc = x_hbm.shape[1] // KG
    per_w = n_sc * SCHUNK
    wid = lax.axis_index("s") * NC + lax.axis_index("c")
    pltpu.sync_copy(x_hbm.at[wid], idx_v)
    base = wid * per_w
    rows = (rows0, rows1)
    outs = (out0, out1)
    gsems = (gsem0, gsem1)
    osems = (osem0, osem1)

    def fire_gathers(sc, b):
        for k in range(KG):
            pltpu.async_copy(
                table_hbm.at[idx_v.at[sc * KG + k]],
                rows[b].at[pl.ds(k * GCHUNK, GCHUNK)], gsems[b])

    def drain_gathers(sc, b):
        for k in range(KG):
            pltpu.make_async_copy(
                table_hbm.at[idx_v.at[sc * KG + k]],
                rows[b].at[pl.ds(k * GCHUNK, GCHUNK)], gsems[b]).wait()

    def out_slice(sc):
        return out_hbm.at[pl.ds(base + sc * SCHUNK, SCHUNK)]

    fire_gathers(0, 0)

    def iter_body(i, carry):
        for b in (0, 1):
            sc = 2 * i + b

            @pl.when(sc + 1 < n_sc)
            def _():
                fire_gathers(sc + 1, 1 - b)

            drain_gathers(sc, b)

            @pl.when(sc >= 2)
            def _():
                pltpu.make_async_copy(outs[b], out_slice(sc - 2), osems[b]).wait()

            def group_body(g, carry2):
                row_idx = g * L + lax.iota(jnp.int32, L)
                cols = []
                s = jnp.zeros((L,), jnp.float32)
                sq = jnp.zeros((L,), jnp.float32)
                for j in range(D):
                    v = plsc.load_gather(
                        rows[b], [row_idx, jnp.full((L,), j, jnp.int32)])
                    cols.append(v)
                    s = s + v
                    sq = sq + v * v
                mean = s * (1.0 / D)
                var = sq * (1.0 / D) - mean * mean
                rstd = _rsqrt(var + 1e-5)
                for j in range(D):
                    y = (cols[j] - mean) * rstd
                    plsc.store_scatter(
                        outs[b], [row_idx, jnp.full((L,), j, jnp.int32)], y)
                return carry2

            lax.fori_loop(0, GROUPS, group_body, 0)
            pltpu.async_copy(outs[b], out_slice(sc), osems[b])
        return carry

    lax.fori_loop(0, n_sc // 2, iter_body, 0)
    pltpu.make_async_copy(outs[0], out_slice(n_sc - 2), osems[0]).wait()
    pltpu.make_async_copy(outs[1], out_slice(n_sc - 1), osems[1]).wait()


def kernel(x, table):
    batch, fields = x.shape
    total = batch * fields
    n_g = total // (NW * GCHUNK)
    x_re = x.reshape(NW, n_g, GCHUNK).astype(jnp.int32)
    mesh = plsc.VectorSubcoreMesh(core_axis_name="c", subcore_axis_name="s")
    f = pl.kernel(
        _body,
        mesh=mesh,
        out_type=jax.ShapeDtypeStruct((total, D), jnp.float32),
        scratch_types=[
            pltpu.VMEM((n_g, GCHUNK), jnp.int32),
            pltpu.VMEM((SCHUNK, D), jnp.float32),
            pltpu.VMEM((SCHUNK, D), jnp.float32),
            pltpu.VMEM((SCHUNK, D), jnp.float32),
            pltpu.VMEM((SCHUNK, D), jnp.float32),
            pltpu.SemaphoreType.DMA,
            pltpu.SemaphoreType.DMA,
            pltpu.SemaphoreType.DMA,
            pltpu.SemaphoreType.DMA,
        ],
        compiler_params=pltpu.CompilerParams(
            needs_layout_passes=False, use_tc_tiling_on_sc=False),
    )
    out = f(x_re, table)
    return out.reshape(batch, fields, D)


# speedup vs baseline: 1.7220x; 1.3590x over previous
import jax
import jax.numpy as jnp
from jax import lax
from jax.experimental import pallas as pl
from jax.experimental.pallas import tpu as pltpu, tpu_sc as plsc

D = 32
NC = 2
NS = 16
L = 16
NW = NC * NS
GCHUNK = 128
KG = 4
SCHUNK = GCHUNK * KG
GROUPS = SCHUNK // L


def _rsqrt(x):
    i = plsc.bitcast(x, jnp.int32)
    i = jnp.int32(0x5F3759DF) - (i >> 1)
    y = plsc.bitcast(i, jnp.float32)
    for _ in range(3):
        y = y * (1.5 - 0.5 * x * y * y)
    return y


def _body(x_hbm, table_hbm, out_hbm, idx_v, rows0, rows1, out0, out1,
          gsem0, gsem1, osem0, osem1):
    n_sc = x_hbm.shape[1] // KG
    per_w = n_sc * SCHUNK
    wid = lax.axis_index("s") * NC + lax.axis_index("c")
    pltpu.sync_copy(x_hbm.at[wid], idx_v)
    base = wid * per_w
    rows = (rows0, rows1)
    outs = (out0, out1)
    gsems = (gsem0, gsem1)
    osems = (osem0, osem1)

    def fire_gathers(sc, b):
        for k in range(KG):
            pltpu.async_copy(
                table_hbm.at[idx_v.at[sc * KG + k]],
                rows[b].at[pl.ds(k * GCHUNK, GCHUNK)], gsems[b])

    def drain_gathers(sc, b):
        for k in range(KG):
            pltpu.make_async_copy(
                table_hbm.at[idx_v.at[sc * KG + k]],
                rows[b].at[pl.ds(k * GCHUNK, GCHUNK)], gsems[b]).wait()

    def out_slice(sc):
        return out_hbm.at[pl.ds(base + sc * SCHUNK, SCHUNK)]

    fire_gathers(0, 0)

    def iter_body(i, carry):
        for b in (0, 1):
            sc = 2 * i + b

            @pl.when(sc + 1 < n_sc)
            def _():
                fire_gathers(sc + 1, 1 - b)

            drain_gathers(sc, b)

            @pl.when(sc >= 2)
            def _():
                pltpu.make_async_copy(outs[b], out_slice(sc - 2), osems[b]).wait()

            def group_body(g, carry2):
                lane = lax.iota(jnp.int32, L)
                row_idx = g * L + lane
                cols = []
                s = jnp.zeros((L,), jnp.float32)
                sq = jnp.zeros((L,), jnp.float32)
                for j in range(D):
                    colv = (lane + j) & (D - 1)
                    v = plsc.load_gather(rows[b], [row_idx, colv])
                    cols.append(v)
                    s = s + v
                    sq = sq + v * v
                mean = s * (1.0 / D)
                var = sq * (1.0 / D) - mean * mean
                rstd = _rsqrt(var + 1e-5)
                for j in range(D):
                    y = (cols[j] - mean) * rstd
                    plsc.store_scatter(
                        outs[b], [row_idx, (lane + j) & (D - 1)], y)
                return carry2

            lax.fori_loop(0, GROUPS, group_body, 0)
            pltpu.async_copy(outs[b], out_slice(sc), osems[b])
        return carry

    lax.fori_loop(0, n_sc // 2, iter_body, 0)
    pltpu.make_async_copy(outs[0], out_slice(n_sc - 2), osems[0]).wait()
    pltpu.make_async_copy(outs[1], out_slice(n_sc - 1), osems[1]).wait()


def kernel(x, table):
    batch, fields = x.shape
    total = batch * fields
    n_g = total // (NW * GCHUNK)
    x_re = x.reshape(NW, n_g, GCHUNK).astype(jnp.int32)
    mesh = plsc.VectorSubcoreMesh(core_axis_name="c", subcore_axis_name="s")
    f = pl.kernel(
        _body,
        mesh=mesh,
        out_type=jax.ShapeDtypeStruct((total, D), jnp.float32),
        scratch_types=[
            pltpu.VMEM((n_g, GCHUNK), jnp.int32),
            pltpu.VMEM((SCHUNK, D), jnp.float32),
            pltpu.VMEM((SCHUNK, D), jnp.float32),
            pltpu.VMEM((SCHUNK, D), jnp.float32),
            pltpu.VMEM((SCHUNK, D), jnp.float32),
            pltpu.SemaphoreType.DMA,
            pltpu.SemaphoreType.DMA,
            pltpu.SemaphoreType.DMA,
            pltpu.SemaphoreType.DMA,
        ],
        compiler_params=pltpu.CompilerParams(
            needs_layout_passes=False, use_tc_tiling_on_sc=False),
    )
    out = f(x_re, table)
    return out.reshape(batch, fields, D)
